# Initial kernel scaffold; baseline (speedup 1.0000x reference)
import jax
import jax.numpy as jnp
from jax.experimental import pallas as pl


def kernel(y, A):
    raise NotImplementedError("write your pallas kernel here")


# speedup vs baseline: 1.5101x; 1.5101x over previous
import jax
import jax.numpy as jnp
from jax.experimental import pallas as pl
from jax.experimental.pallas import tpu as pltpu

_NCLS = 10
_B = 1024
_GB = 8
_GRID = _B // _GB


def _body(y_sp, a_ref, o_ref):
    i = pl.program_id(0)
    for j in range(_GB):
        row = y_sp[i * _GB + j]
        o_ref[j] = a_ref[row]


def kernel(y, A):
    a3 = A.reshape(_NCLS, 100, 784)
    out = pl.pallas_call(
        _body,
        grid_spec=pltpu.PrefetchScalarGridSpec(
            num_scalar_prefetch=1,
            grid=(_GRID,),
            in_specs=[pl.BlockSpec((_NCLS, 100, 784), lambda i, y_sp: (0, 0, 0))],
            out_specs=pl.BlockSpec((_GB, 100, 784), lambda i, y_sp: (i, 0, 0)),
        ),
        out_shape=jax.ShapeDtypeStruct((_B, 100, 784), jnp.float32),
        compiler_params=pltpu.CompilerParams(dimension_semantics=("arbitrary",)),
    )(y.astype(jnp.int32), a3)
    return out


# speedup vs baseline: 1.5206x; 1.0070x over previous
import jax
import jax.numpy as jnp
from jax import lax
from jax.experimental import pallas as pl
from jax.experimental.pallas import tpu as pltpu

_NCLS = 10
_B = 1024
_K = 16


def _body(y_sp, a_ref, o_ref, sems):
    def start(i):
        pltpu.make_async_copy(a_ref.at[y_sp[i]], o_ref.at[i],
                              sems.at[i % _K]).start()

    def wait(i):
        pltpu.make_async_copy(a_ref.at[0], o_ref.at[i],
                              sems.at[i % _K]).wait()

    for i in range(_K):
        start(i)

    def loop(i, _):
        wait(i - _K)
        start(i)
        return ()

    lax.fori_loop(_K, _B, loop, ())

    for i in range(_B - _K, _B):
        wait(i)


def kernel(y, A):
    a3 = A.reshape(_NCLS, 100, 784)
    out = pl.pallas_call(
        _body,
        grid_spec=pltpu.PrefetchScalarGridSpec(
            num_scalar_prefetch=1,
            grid=(1,),
            in_specs=[pl.BlockSpec((_NCLS, 100, 784), lambda i, y_sp: (0, 0, 0))],
            out_specs=pl.BlockSpec(memory_space=pl.ANY),
            scratch_shapes=[pltpu.SemaphoreType.DMA((_K,))],
        ),
        out_shape=jax.ShapeDtypeStruct((_B, 100, 784), jnp.float32),
        compiler_params=pltpu.CompilerParams(dimension_semantics=("arbitrary",)),
    )(y.astype(jnp.int32), a3)
    return out
